# baseline (speedup 1.0000x reference)
import functools

import jax
import jax.numpy as jnp
from jax import lax
from jax.experimental import pallas as pl
from jax.experimental.pallas import tpu as pltpu
from jax.experimental.pallas import tpu_sc as plsc

SIGMA = 0.7
EPS = 1e-5
LEAKY = 0.1


def _make_sc_gather(N, C, B, NW, CH, UNROLL):
    NCH = B // (NW * CH)
    assert B == NW * NCH * CH and NCH % UNROLL == 0
    mesh = plsc.VectorSubcoreMesh(core_axis_name="c", subcore_axis_name="s")
    NC = 2

    @functools.partial(
        pl.kernel,
        mesh=mesh,
        compiler_params=pltpu.CompilerParams(use_tc_tiling_on_sc=False),
        out_type=[
            jax.ShapeDtypeStruct((B, C), jnp.bfloat16),
            jax.ShapeDtypeStruct((B, 16), jnp.float32),
        ],
        scratch_types=(
            [pltpu.VMEM((NCH, CH), jnp.int32),
             pltpu.VMEM((UNROLL, CH, C), jnp.bfloat16),
             pltpu.VMEM((UNROLL, CH, 16), jnp.float32)]
            + [pltpu.SemaphoreType.DMA] * (2 * UNROLL)
        ),
    )
    def sc_gather(feats_hbm, pts_hbm, idx_hbm, gf_out, gp_out,
                  idx_v, fbuf, pbuf, *sems):
        semf = sems[:UNROLL]
        semp = sems[UNROLL:]
        wid = lax.axis_index("s") * NC + lax.axis_index("c")
        base = wid * (NCH * CH)
        pltpu.sync_copy(idx_hbm.at[wid], idx_v)

        def body(t, _):
            j = t * UNROLL
            cps = []
            for b in range(UNROLL):
                cf = pltpu.async_copy(feats_hbm.at[idx_v.at[j + b]],
                                      fbuf.at[b], semf[b])
                cp = pltpu.async_copy(pts_hbm.at[idx_v.at[j + b]],
                                      pbuf.at[b], semp[b])
                cps.append((cf, cp))
            for b in range(UNROLL):
                off = base + (j + b) * CH
                cf, cp = cps[b]
                cf.wait()
                pltpu.sync_copy(fbuf.at[b], gf_out.at[pl.ds(off, CH)])
                cp.wait()
                pltpu.sync_copy(pbuf.at[b], gp_out.at[pl.ds(off, CH)])
            return 0

        lax.fori_loop(0, NCH // UNROLL, body, 0)

    return sc_gather


def _t1_body(gp_ref, gf_ref, qp_ref, ck_ref, wr_ref, out_ref, stats_ref,
             wf_ref, *, TM, H, C, K, C_OUT):
    i = pl.program_id(0)
    NS = TM // 8
    E = TM * H
    gp3 = gp_ref[...].reshape(TM, H, 16)
    rel = (gp3 - qp_ref[...][:, None, :]).reshape(E, 16)
    rsq = jnp.sum(rel * rel, axis=1, keepdims=True)
    lane = lax.broadcasted_iota(jnp.int32, (1, 16), 1)
    r16 = (rel + jnp.where(lane == 3, rsq, 0.0)
           + jnp.where(lane == 4, 1.0, 0.0))
    d2 = lax.dot_general(r16, ck_ref[...], (((1,), (0,)), ((), ())),
                         precision=lax.Precision.HIGHEST,
                         preferred_element_type=jnp.float32)
    d2 = jnp.maximum(d2, 0.0)
    we = jnp.maximum(1.0 - jnp.sqrt(d2) * (1.0 / SIGMA), 0.0)
    wt = we.T
    r0 = lax.broadcasted_iota(jnp.int32, (128, 8 * H), 0)
    c0 = lax.broadcasted_iota(jnp.int32, (128, 8 * H), 1)
    mask = (r0 % 8) == (c0 // H)
    for s in range(NS):
        ws = wt[:, s * 8 * H:(s + 1) * 8 * H]
        rep = jnp.broadcast_to(ws[:, None, :],
                               (16, 8, 8 * H)).reshape(128, 8 * H)
        bd = jnp.where(mask, rep, 0.0)
        gfs = gf_ref[s * 8 * H:(s + 1) * 8 * H, :]
        y = lax.dot_general(bd.astype(jnp.bfloat16), gfs,
                            (((1,), (0,)), ((), ())),
                            preferred_element_type=jnp.float32)
        for k in range(K):
            wf_ref[k * TM + s * 8:k * TM + s * 8 + 8, :] = \
                y[k * 8:(k + 1) * 8, :]
    acc = jnp.zeros((TM, C_OUT), jnp.float32)
    for k in range(K):
        acc = acc + lax.dot_general(
            wf_ref[k * TM:(k + 1) * TM, :],
            wr_ref[k * C:(k + 1) * C, :],
            (((1,), (0,)), ((), ())), preferred_element_type=jnp.float32)
    out_ref[...] = acc
    ps = jnp.sum(acc, axis=0, keepdims=True)
    pss = jnp.sum(acc * acc, axis=0, keepdims=True)
    part = jnp.concatenate(
        [ps, pss, jnp.zeros((6, C_OUT), jnp.float32)], axis=0)

    @pl.when(i == 0)
    def _():
        stats_ref[...] = part

    @pl.when(i > 0)
    def _():
        stats_ref[...] = stats_ref[...] + part


def _t2_body(x_ref, stats_ref, g_ref, b_ref, o_ref, *, M):
    x = x_ref[...]
    mean = stats_ref[0:1, :] * (1.0 / M)
    var = stats_ref[1:2, :] * (1.0 / M) - mean * mean
    scale = lax.rsqrt(var + EPS) * g_ref[...]
    y = (x - mean) * scale + b_ref[...]
    o_ref[...] = jnp.where(y >= 0, y, LEAKY * y)


def kernel(q_pts, s_pts, s_feats, neighb_inds, kernel_points, weights,
           gamma, beta):
    M, H = neighb_inds.shape
    N, C = s_feats.shape
    K = kernel_points.shape[0]
    C_OUT = weights.shape[2]
    B = M * H
    NW, CH, UNROLL = 32, 80, 5
    NCH = B // (NW * CH)
    TM = 200
    GRID = M // TM

    pts16 = jnp.pad(s_pts, ((0, 0), (0, 13)))
    qp16 = jnp.pad(q_pts, ((0, 0), (0, 13)))
    ck = jnp.concatenate(
        [-2.0 * kernel_points.T,
         jnp.ones((1, K), jnp.float32),
         jnp.sum(kernel_points * kernel_points, axis=1)[None, :],
         jnp.zeros((11, K), jnp.float32)], axis=0)
    ck = jnp.pad(ck, ((0, 0), (0, 16 - K)))
    w_r = weights.reshape(K * C, C_OUT)
    idx3 = neighb_inds.reshape(NW, NCH, CH)

    gf, gp = _make_sc_gather(N, C, B, NW, CH, UNROLL)(
        s_feats.astype(jnp.bfloat16), pts16, idx3)

    t1 = pl.pallas_call(
        functools.partial(_t1_body, TM=TM, H=H, C=C, K=K, C_OUT=C_OUT),
        grid=(GRID,),
        in_specs=[
            pl.BlockSpec((TM * H, 16), lambda i: (i, 0)),
            pl.BlockSpec((TM * H, C), lambda i: (i, 0)),
            pl.BlockSpec((TM, 16), lambda i: (i, 0)),
            pl.BlockSpec((16, 16), lambda i: (0, 0)),
            pl.BlockSpec((K * C, C_OUT), lambda i: (0, 0)),
        ],
        out_specs=[
            pl.BlockSpec((TM, C_OUT), lambda i: (i, 0)),
            pl.BlockSpec((8, C_OUT), lambda i: (0, 0)),
        ],
        out_shape=[
            jax.ShapeDtypeStruct((M, C_OUT), jnp.float32),
            jax.ShapeDtypeStruct((8, C_OUT), jnp.float32),
        ],
        scratch_shapes=[pltpu.VMEM((K * TM, C_OUT), jnp.float32)],
        compiler_params=pltpu.CompilerParams(
            dimension_semantics=("arbitrary",)),
    )
    out_pre, stats = t1(gp, gf, qp16, ck, w_r)

    t2 = pl.pallas_call(
        functools.partial(_t2_body, M=M),
        grid=(GRID,),
        in_specs=[
            pl.BlockSpec((TM, C_OUT), lambda i: (i, 0)),
            pl.BlockSpec((8, C_OUT), lambda i: (0, 0)),
            pl.BlockSpec((1, C_OUT), lambda i: (0, 0)),
            pl.BlockSpec((1, C_OUT), lambda i: (0, 0)),
        ],
        out_specs=pl.BlockSpec((TM, C_OUT), lambda i: (i, 0)),
        out_shape=jax.ShapeDtypeStruct((M, C_OUT), jnp.float32),
    )
    return t2(out_pre, stats, gamma.reshape(1, C_OUT), beta.reshape(1, C_OUT))


# speedup vs baseline: 1.6265x; 1.6265x over previous
import functools

import jax
import jax.numpy as jnp
from jax import lax
from jax.experimental import pallas as pl
from jax.experimental.pallas import tpu as pltpu
from jax.experimental.pallas import tpu_sc as plsc

SIGMA = 0.7
EPS = 1e-5
LEAKY = 0.1


def _make_sc_gather(N, C, B, NW, CH, UNROLL):
    NCH = B // (NW * CH)
    assert B == NW * NCH * CH and NCH % UNROLL == 0
    mesh = plsc.VectorSubcoreMesh(core_axis_name="c", subcore_axis_name="s")
    NC = 2

    @functools.partial(
        pl.kernel,
        mesh=mesh,
        compiler_params=pltpu.CompilerParams(use_tc_tiling_on_sc=False),
        out_type=[
            jax.ShapeDtypeStruct((B, C), jnp.float32),
            jax.ShapeDtypeStruct((B, 16), jnp.float32),
        ],
        scratch_types=(
            [pltpu.VMEM((NCH, CH), jnp.int32),
             pltpu.VMEM((UNROLL, CH, C), jnp.float32),
             pltpu.VMEM((UNROLL, CH, 16), jnp.float32)]
            + [pltpu.SemaphoreType.DMA] * (2 * UNROLL)
        ),
    )
    def sc_gather(feats_hbm, pts_hbm, idx_hbm, gf_out, gp_out,
                  idx_v, fbuf, pbuf, *sems):
        semf = sems[:UNROLL]
        semp = sems[UNROLL:]
        wid = lax.axis_index("s") * NC + lax.axis_index("c")
        base = wid * (NCH * CH)
        pltpu.sync_copy(idx_hbm.at[wid], idx_v)

        def body(t, _):
            j = t * UNROLL
            cps = []
            for b in range(UNROLL):
                cf = pltpu.async_copy(feats_hbm.at[idx_v.at[j + b]],
                                      fbuf.at[b], semf[b])
                cp = pltpu.async_copy(pts_hbm.at[idx_v.at[j + b]],
                                      pbuf.at[b], semp[b])
                cps.append((cf, cp))
            for b in range(UNROLL):
                off = base + (j + b) * CH
                cf, cp = cps[b]
                cf.wait()
                pltpu.sync_copy(fbuf.at[b], gf_out.at[pl.ds(off, CH)])
                cp.wait()
                pltpu.sync_copy(pbuf.at[b], gp_out.at[pl.ds(off, CH)])
            return 0

        lax.fori_loop(0, NCH // UNROLL, body, 0)

    return sc_gather


def _t1_body(gp_ref, gf_ref, qp_ref, ck_ref, wr_ref, out_ref, stats_ref,
             wf_ref, *, TM, H, C, K, C_OUT):
    i = pl.program_id(0)
    NS = TM // 8
    E = TM * H
    gp3 = gp_ref[...].reshape(TM, H, 16)
    rel = (gp3 - qp_ref[...][:, None, :]).reshape(E, 16)
    rsq = jnp.sum(rel * rel, axis=1, keepdims=True)
    lane = lax.broadcasted_iota(jnp.int32, (1, 16), 1)
    r16 = (rel + jnp.where(lane == 3, rsq, 0.0)
           + jnp.where(lane == 4, 1.0, 0.0))
    d2 = lax.dot_general(r16, ck_ref[...], (((1,), (0,)), ((), ())),
                         precision=lax.Precision.HIGHEST,
                         preferred_element_type=jnp.float32)
    d2 = jnp.maximum(d2, 0.0)
    we = jnp.maximum(1.0 - jnp.sqrt(d2) * (1.0 / SIGMA), 0.0)
    wt = we.T
    r0 = lax.broadcasted_iota(jnp.int32, (128, 8 * H), 0)
    c0 = lax.broadcasted_iota(jnp.int32, (128, 8 * H), 1)
    mask = (r0 % 8) == (c0 // H)
    for s in range(NS):
        ws = wt[:, s * 8 * H:(s + 1) * 8 * H]
        rep = jnp.broadcast_to(ws[:, None, :],
                               (16, 8, 8 * H)).reshape(128, 8 * H)
        bd = jnp.where(mask, rep, 0.0)
        gfs = gf_ref[s * 8 * H:(s + 1) * 8 * H, :]
        y = lax.dot_general(bd, gfs, (((1,), (0,)), ((), ())),
                            preferred_element_type=jnp.float32)
        for k in range(K):
            wf_ref[k * TM + s * 8:k * TM + s * 8 + 8, :] = \
                y[k * 8:(k + 1) * 8, :]
    acc = jnp.zeros((TM, C_OUT), jnp.float32)
    for k in range(K):
        acc = acc + lax.dot_general(
            wf_ref[k * TM:(k + 1) * TM, :],
            wr_ref[k * C:(k + 1) * C, :],
            (((1,), (0,)), ((), ())), preferred_element_type=jnp.float32)
    out_ref[...] = acc
    ps = jnp.sum(acc, axis=0, keepdims=True)
    pss = jnp.sum(acc * acc, axis=0, keepdims=True)
    part = jnp.concatenate(
        [ps, pss, jnp.zeros((6, C_OUT), jnp.float32)], axis=0)

    @pl.when(i == 0)
    def _():
        stats_ref[...] = part

    @pl.when(i > 0)
    def _():
        stats_ref[...] = stats_ref[...] + part


def _t2_body(x_ref, stats_ref, g_ref, b_ref, o_ref, *, M):
    x = x_ref[...]
    mean = stats_ref[0:1, :] * (1.0 / M)
    var = stats_ref[1:2, :] * (1.0 / M) - mean * mean
    scale = lax.rsqrt(var + EPS) * g_ref[...]
    y = (x - mean) * scale + b_ref[...]
    o_ref[...] = jnp.where(y >= 0, y, LEAKY * y)


def kernel(q_pts, s_pts, s_feats, neighb_inds, kernel_points, weights,
           gamma, beta):
    M, H = neighb_inds.shape
    N, C = s_feats.shape
    K = kernel_points.shape[0]
    C_OUT = weights.shape[2]
    B = M * H
    NW, CH, UNROLL = 32, 80, 5
    NCH = B // (NW * CH)
    TM = 400
    GRID = M // TM

    pts16 = jnp.pad(s_pts, ((0, 0), (0, 13)))
    qp16 = jnp.pad(q_pts, ((0, 0), (0, 13)))
    ck = jnp.concatenate(
        [-2.0 * kernel_points.T,
         jnp.ones((1, K), jnp.float32),
         jnp.sum(kernel_points * kernel_points, axis=1)[None, :],
         jnp.zeros((11, K), jnp.float32)], axis=0)
    ck = jnp.pad(ck, ((0, 0), (0, 16 - K)))
    w_r = weights.reshape(K * C, C_OUT)
    idx3 = neighb_inds.reshape(NW, NCH, CH)

    gf, gp = _make_sc_gather(N, C, B, NW, CH, UNROLL)(s_feats, pts16, idx3)

    t1 = pl.pallas_call(
        functools.partial(_t1_body, TM=TM, H=H, C=C, K=K, C_OUT=C_OUT),
        grid=(GRID,),
        in_specs=[
            pl.BlockSpec((TM * H, 16), lambda i: (i, 0)),
            pl.BlockSpec((TM * H, C), lambda i: (i, 0)),
            pl.BlockSpec((TM, 16), lambda i: (i, 0)),
            pl.BlockSpec((16, 16), lambda i: (0, 0)),
            pl.BlockSpec((K * C, C_OUT), lambda i: (0, 0)),
        ],
        out_specs=[
            pl.BlockSpec((TM, C_OUT), lambda i: (i, 0)),
            pl.BlockSpec((8, C_OUT), lambda i: (0, 0)),
        ],
        out_shape=[
            jax.ShapeDtypeStruct((M, C_OUT), jnp.float32),
            jax.ShapeDtypeStruct((8, C_OUT), jnp.float32),
        ],
        scratch_shapes=[pltpu.VMEM((K * TM, C_OUT), jnp.float32)],
        compiler_params=pltpu.CompilerParams(
            dimension_semantics=("arbitrary",)),
    )
    out_pre, stats = t1(gp, gf, qp16, ck, w_r)

    t2 = pl.pallas_call(
        functools.partial(_t2_body, M=M),
        grid=(GRID,),
        in_specs=[
            pl.BlockSpec((TM, C_OUT), lambda i: (i, 0)),
            pl.BlockSpec((8, C_OUT), lambda i: (0, 0)),
            pl.BlockSpec((1, C_OUT), lambda i: (0, 0)),
            pl.BlockSpec((1, C_OUT), lambda i: (0, 0)),
        ],
        out_specs=pl.BlockSpec((TM, C_OUT), lambda i: (i, 0)),
        out_shape=jax.ShapeDtypeStruct((M, C_OUT), jnp.float32),
    )
    return t2(out_pre, stats, gamma.reshape(1, C_OUT), beta.reshape(1, C_OUT))
